# baseline (speedup 1.0000x reference)
import jax
import jax.numpy as jnp
from jax.experimental import pallas as pl
from jax.experimental.pallas import tpu as pltpu


def _gconv_block_kernel(flt_ref, x_ref, w_ref, o_ref):
    acc = jnp.dot(flt_ref[...], x_ref[...], preferred_element_type=jnp.float32)
    lin = jax.lax.dot_general(
        acc, w_ref[...], (((1,), (1,)), ((), ())),
        preferred_element_type=jnp.float32)
    o_ref[...] = jnp.tanh(lin)


def _pick_block_rows(n_rows: int) -> int:
    for bm in (200, 80, 40, 16, 8):
        if n_rows % bm == 0:
            return bm
    return n_rows


def kernel(inputs, flt, W):
    n_rows, n_cols = flt.shape
    d_in = inputs.shape[1]
    d_out = W.shape[0]
    bm = _pick_block_rows(n_rows)
    return pl.pallas_call(
        _gconv_block_kernel,
        grid=(n_rows // bm,),
        in_specs=[
            pl.BlockSpec((bm, n_cols), lambda i: (i, 0)),
            pl.BlockSpec((n_cols, d_in), lambda i: (0, 0)),
            pl.BlockSpec((d_out, d_in), lambda i: (0, 0)),
        ],
        out_specs=pl.BlockSpec((bm, d_out), lambda i: (i, 0)),
        out_shape=jax.ShapeDtypeStruct((n_rows, d_out), jnp.float32),
        compiler_params=pltpu.CompilerParams(
            dimension_semantics=("arbitrary",)),
    )(flt, inputs, W)


# speedup vs baseline: 1.0286x; 1.0286x over previous
import jax
import jax.numpy as jnp
from jax.experimental import pallas as pl
from jax.experimental.pallas import tpu as pltpu


def _gconv_block_kernel(flt_ref, x_ref, w_ref, o_ref):
    acc = jnp.dot(flt_ref[...], x_ref[...], preferred_element_type=jnp.float32)
    lin = jax.lax.dot_general(
        acc, w_ref[...], (((1,), (1,)), ((), ())),
        preferred_element_type=jnp.float32)
    o_ref[...] = jnp.tanh(lin)


def _pick_block_rows(n_rows: int) -> int:
    for bm in (400, 200, 80, 40, 16, 8):
        if n_rows % bm == 0:
            return bm
    return n_rows


def kernel(inputs, flt, W):
    n_rows, n_cols = flt.shape
    d_in = inputs.shape[1]
    d_out = W.shape[0]
    bm = _pick_block_rows(n_rows)
    return pl.pallas_call(
        _gconv_block_kernel,
        grid=(n_rows // bm,),
        in_specs=[
            pl.BlockSpec((bm, n_cols), lambda i: (i, 0)),
            pl.BlockSpec((n_cols, d_in), lambda i: (0, 0)),
            pl.BlockSpec((d_out, d_in), lambda i: (0, 0)),
        ],
        out_specs=pl.BlockSpec((bm, d_out), lambda i: (i, 0)),
        out_shape=jax.ShapeDtypeStruct((n_rows, d_out), jnp.float32),
        compiler_params=pltpu.CompilerParams(
            dimension_semantics=("arbitrary",)),
    )(flt, inputs, W)


# speedup vs baseline: 1.0287x; 1.0001x over previous
import jax
import jax.numpy as jnp
from jax.experimental import pallas as pl
from jax.experimental.pallas import tpu as pltpu


def _gconv_block_kernel(flt_ref, x_ref, w_ref, o_ref):
    acc = jnp.dot(flt_ref[...], x_ref[...], preferred_element_type=jnp.float32)
    lin = jax.lax.dot_general(
        acc, w_ref[...], (((1,), (1,)), ((), ())),
        preferred_element_type=jnp.float32)
    o_ref[...] = jnp.tanh(lin)


def _pick_block_rows(n_rows: int) -> int:
    for bm in (400, 200, 80, 40, 16, 8):
        if n_rows % bm == 0:
            return bm
    return n_rows


def kernel(inputs, flt, W):
    n_rows, n_cols = flt.shape
    d_in = inputs.shape[1]
    d_out = W.shape[0]
    bm = _pick_block_rows(n_rows)
    return pl.pallas_call(
        _gconv_block_kernel,
        grid=(n_rows // bm,),
        in_specs=[
            pl.BlockSpec((bm, n_cols), lambda i: (i, 0)),
            pl.BlockSpec((n_cols, d_in), lambda i: (0, 0)),
            pl.BlockSpec((d_out, d_in), lambda i: (0, 0)),
        ],
        out_specs=pl.BlockSpec((bm, d_out), lambda i: (i, 0)),
        out_shape=jax.ShapeDtypeStruct((n_rows, d_out), jnp.float32),
        compiler_params=pltpu.CompilerParams(
            dimension_semantics=("parallel",)),
    )(flt, inputs, W)
